# baseline (speedup 1.0000x reference)
import functools

import jax
import jax.numpy as jnp
from jax.experimental import pallas as pl

DT = 1.0


def _fused(B, N, T, F, H, HOR, E,
           xFT_ref, WallT_ref, b_ref, K_ref, X_ref, src_ref, dst_ref,
           Ws0_ref, Wm0_ref, bl0_ref, Ws1_ref, Wm1_ref, bl1_ref,
           hW1_ref, hb1_ref, hW2_ref, hb2_ref, out_ref):
    BN = B * N
    WallT = WallT_ref[...]

    C = 8
    CH = T // C
    b = b_ref[...]

    def sigm(v):
        return 0.5 * jnp.tanh(0.5 * v) + 0.5

    NL = 2
    LB = BN // NL
    WhhT = WallT[:, 0:H]
    WihT = WallT[:, H:H + F]

    def xproj(t, j):
        xtT = xFT_ref[:, pl.ds(t * BN + j * LB, LB)]
        return b + jnp.dot(WihT, xtT, preferred_element_type=jnp.float32)

    def outer(ci, carry):
        hs, cs, xw = list(carry[0]), list(carry[1]), list(carry[2])
        for k in range(C):
            t = ci * C + k
            for j in range(NL):
                gIF = xw[j][0:2 * H] + jnp.dot(
                    WhhT[0:2 * H], hs[j], preferred_element_type=jnp.float32)
                gGO = xw[j][2 * H:4 * H] + jnp.dot(
                    WhhT[2 * H:4 * H], hs[j],
                    preferred_element_type=jnp.float32)
                xw[j] = xproj(jnp.minimum(t + 1, T - 1), j)
                i = sigm(gIF[0:H])
                f = sigm(gIF[H:2 * H])
                g = jnp.tanh(gGO[0:H])
                o = sigm(gGO[H:2 * H])
                cs[j] = f * cs[j] + i * g
                hs[j] = o * jnp.tanh(cs[j])
        return (tuple(hs), tuple(cs), tuple(xw))

    h0 = tuple(jnp.zeros((H, LB), jnp.float32) for _ in range(NL))
    c0 = tuple(jnp.zeros((H, LB), jnp.float32) for _ in range(NL))
    xw0 = tuple(xproj(0, j) for j in range(NL))
    hs, _, _ = jax.lax.fori_loop(0, CH, outer, (h0, c0, xw0))
    hT = jnp.concatenate(hs, axis=1)
    h = hT.T

    K = K_ref[...]
    X = X_ref[...]
    denom = K - K * X + 0.5 * DT
    w = ((-K * X + 0.5 * DT) / denom) + ((K * X + 0.5 * DT) / denom)
    node_ids = jax.lax.broadcasted_iota(jnp.int32, (N, E), 0)
    src_oh = src_ref[...] == node_ids
    dst_oh = dst_ref[...] == node_ids
    wsrc = jnp.where(src_oh, w, 0.0)
    dstf = jnp.where(dst_oh, 1.0, 0.0)
    A = jax.lax.dot_general(dstf, wsrc, (((1,), (1,)), ((), ())),
                            preferred_element_type=jnp.float32)

    def graph_layer(hcur, Ws, Wm, bl):
        hWs = jnp.dot(hcur, Ws, preferred_element_type=jnp.float32)
        aggs = []
        for bi in range(B):
            hb = hcur[bi * N:(bi + 1) * N]
            aggs.append(jnp.dot(A, hb, preferred_element_type=jnp.float32))
        agg = jnp.concatenate(aggs, axis=0)
        z = hWs + jnp.dot(agg, Wm, preferred_element_type=jnp.float32) + bl
        return jax.nn.gelu(z) + hcur

    h = graph_layer(h, Ws0_ref[...], Wm0_ref[...], bl0_ref[...])
    h = graph_layer(h, Ws1_ref[...], Wm1_ref[...], bl1_ref[...])

    z = jax.nn.gelu(jnp.dot(h, hW1_ref[...], preferred_element_type=jnp.float32)
                    + hb1_ref[...])
    out_ref[...] = (jnp.dot(z, hW2_ref[...], preferred_element_type=jnp.float32)
                    + hb2_ref[...])


def kernel(x, Wih, Whh, b_lstm, K, X, Ws0, Wm0, bl0, Ws1, Wm1, bl1,
           hW1, hb1, hW2, hb2, edge_index):
    B, N, T, F = x.shape
    H = Whh.shape[0]
    HOR = hW2.shape[1]
    E = edge_index.shape[1]
    BN = B * N

    xFT = jnp.transpose(x.reshape(BN, T, F), (2, 1, 0)).reshape(F, T * BN)
    src = edge_index[0:1, :]
    dst = edge_index[1:2, :]
    WallT = jnp.concatenate([Whh, Wih], axis=0).T
    args = (xFT, WallT, b_lstm.reshape(4 * H, 1), K.reshape(1, E),
            X.reshape(1, E), src, dst, Ws0, Wm0, bl0.reshape(1, H),
            Ws1, Wm1, bl1.reshape(1, H), hW1, hb1.reshape(1, H),
            hW2, hb2.reshape(1, HOR))

    out = pl.pallas_call(
        functools.partial(_fused, B, N, T, F, H, HOR, E),
        out_shape=jax.ShapeDtypeStruct((BN, HOR), jnp.float32),
    )(*args)
    return out.reshape(B, N, HOR)


# speedup vs baseline: 1.0214x; 1.0214x over previous
import functools

import jax
import jax.numpy as jnp
from jax.experimental import pallas as pl

DT = 1.0


def _fused(B, N, T, F, H, HOR, E,
           xFT_ref, WallT_ref, b_ref, K_ref, X_ref, src_ref, dst_ref,
           Ws0_ref, Wm0_ref, bl0_ref, Ws1_ref, Wm1_ref, bl1_ref,
           hW1_ref, hb1_ref, hW2_ref, hb2_ref, out_ref):
    BN = B * N
    WallT = WallT_ref[...]

    C = 16
    CH = T // C
    b = b_ref[...]

    def sigm(v):
        return 0.5 * jnp.tanh(0.5 * v) + 0.5

    NL = 2
    LB = BN // NL
    WhhT = WallT[:, 0:H]
    WihT = WallT[:, H:H + F]

    def xproj(t, j):
        xtT = xFT_ref[:, pl.ds(t * BN + j * LB, LB)]
        return b + jnp.dot(WihT, xtT, preferred_element_type=jnp.float32)

    def outer(ci, carry):
        hs, cs, xw = list(carry[0]), list(carry[1]), list(carry[2])
        for k in range(C):
            t = ci * C + k
            for j in range(NL):
                gT = xw[j] + jnp.dot(WhhT, hs[j],
                                     preferred_element_type=jnp.float32)
                xw[j] = xproj(jnp.minimum(t + 1, T - 1), j)
                i = sigm(gT[0:H])
                f = sigm(gT[H:2 * H])
                g = jnp.tanh(gT[2 * H:3 * H])
                o = sigm(gT[3 * H:4 * H])
                cs[j] = f * cs[j] + i * g
                hs[j] = o * jnp.tanh(cs[j])
        return (tuple(hs), tuple(cs), tuple(xw))

    h0 = tuple(jnp.zeros((H, LB), jnp.float32) for _ in range(NL))
    c0 = tuple(jnp.zeros((H, LB), jnp.float32) for _ in range(NL))
    xw0 = tuple(xproj(0, j) for j in range(NL))
    hs, _, _ = jax.lax.fori_loop(0, CH, outer, (h0, c0, xw0))
    hT = jnp.concatenate(hs, axis=1)
    h = hT.T

    K = K_ref[...]
    X = X_ref[...]
    denom = K - K * X + 0.5 * DT
    w = ((-K * X + 0.5 * DT) / denom) + ((K * X + 0.5 * DT) / denom)
    node_ids = jax.lax.broadcasted_iota(jnp.int32, (N, E), 0)
    src_oh = src_ref[...] == node_ids
    dst_oh = dst_ref[...] == node_ids
    wsrc = jnp.where(src_oh, w, 0.0)
    dstf = jnp.where(dst_oh, 1.0, 0.0)
    A = jax.lax.dot_general(dstf, wsrc, (((1,), (1,)), ((), ())),
                            preferred_element_type=jnp.float32)

    def graph_layer(hcur, Ws, Wm, bl):
        hWs = jnp.dot(hcur, Ws, preferred_element_type=jnp.float32)
        aggs = []
        for bi in range(B):
            hb = hcur[bi * N:(bi + 1) * N]
            aggs.append(jnp.dot(A, hb, preferred_element_type=jnp.float32))
        agg = jnp.concatenate(aggs, axis=0)
        z = hWs + jnp.dot(agg, Wm, preferred_element_type=jnp.float32) + bl
        return jax.nn.gelu(z) + hcur

    h = graph_layer(h, Ws0_ref[...], Wm0_ref[...], bl0_ref[...])
    h = graph_layer(h, Ws1_ref[...], Wm1_ref[...], bl1_ref[...])

    z = jax.nn.gelu(jnp.dot(h, hW1_ref[...], preferred_element_type=jnp.float32)
                    + hb1_ref[...])
    out_ref[...] = (jnp.dot(z, hW2_ref[...], preferred_element_type=jnp.float32)
                    + hb2_ref[...])


def kernel(x, Wih, Whh, b_lstm, K, X, Ws0, Wm0, bl0, Ws1, Wm1, bl1,
           hW1, hb1, hW2, hb2, edge_index):
    B, N, T, F = x.shape
    H = Whh.shape[0]
    HOR = hW2.shape[1]
    E = edge_index.shape[1]
    BN = B * N

    xFT = jnp.transpose(x.reshape(BN, T, F), (2, 1, 0)).reshape(F, T * BN)
    src = edge_index[0:1, :]
    dst = edge_index[1:2, :]
    WallT = jnp.concatenate([Whh, Wih], axis=0).T
    args = (xFT, WallT, b_lstm.reshape(4 * H, 1), K.reshape(1, E),
            X.reshape(1, E), src, dst, Ws0, Wm0, bl0.reshape(1, H),
            Ws1, Wm1, bl1.reshape(1, H), hW1, hb1.reshape(1, H),
            hW2, hb2.reshape(1, HOR))

    out = pl.pallas_call(
        functools.partial(_fused, B, N, T, F, H, HOR, E),
        out_shape=jax.ShapeDtypeStruct((BN, HOR), jnp.float32),
    )(*args)
    return out.reshape(B, N, HOR)
